# Initial kernel scaffold; baseline (speedup 1.0000x reference)
import jax
import jax.numpy as jnp
from jax.experimental import pallas as pl


def kernel(pred_logits, pred_boxes, target_sizes):
    raise NotImplementedError("write your pallas kernel here")


# speedup vs baseline: 1.7708x; 1.7708x over previous
import functools
import math

import jax
import jax.numpy as jnp
from jax.experimental import pallas as pl
from jax.experimental.pallas import tpu as pltpu

NSEL = 100
NCAND = 128
NEG = float("-inf")
IBIG = 2**30


def _body(x_ref, box_ref, ts_ref, s_ref, l_ref, b_ref, oh_ref,
          rowmax_ref, cand_ref, base_ref, *, rows_p, n_chunks, c_dim):
  def p1(c, _):
    tile = x_ref[0, pl.ds(c * 128, 128), :]
    rmax = jnp.max(tile, axis=1).reshape(1, 128)
    rowmax_ref[pl.ds(c, 1), :] = rmax
    return 0

  jax.lax.fori_loop(0, n_chunks, p1, 0)

  rid2d = (jax.lax.broadcasted_iota(jnp.int32, (n_chunks, 128), 0) * 128
           + jax.lax.broadcasted_iota(jnp.int32, (n_chunks, 128), 1))

  def p2(k, rm):
    m = jnp.max(rm)
    r = jnp.min(jnp.where(rm == m, rid2d, IBIG))
    cand_ref[pl.ds(k, 1), :] = x_ref[0, pl.ds(r, 1), :]
    base_ref[pl.ds(k, 1), :] = jnp.full((1, 1), r, jnp.int32)
    return jnp.where(rid2d == r, NEG, rm)

  jax.lax.fori_loop(0, NCAND, p2, rowmax_ref[:, :])

  flatid = (base_ref[:, :] * 128
            + jax.lax.broadcasted_iota(jnp.int32, (NCAND, 128), 1))

  img_h = ts_ref[0, 0, 0]
  img_w = ts_ref[0, 0, 1]

  lane128 = jax.lax.broadcasted_iota(jnp.int32, (1, 128), 1)

  def p3(k, cand):
    m = jnp.max(cand)
    p = jnp.min(jnp.where(cand == m, flatid, IBIG))
    score = 1.0 / (1.0 + jnp.exp(-m))
    label = jax.lax.rem(p, jnp.int32(c_dim))
    q = jax.lax.div(p, jnp.int32(c_dim))
    s_ref[0, pl.ds(k, 1), :] = jnp.full((1, 1), score, jnp.float32)
    l_ref[0, pl.ds(k, 1), :] = jnp.full((1, 1), label, jnp.int32)
    oh_ref[0, pl.ds(k, 1), :] = jnp.where(lane128 == label, 1.0, 0.0)
    row16 = box_ref[0, pl.ds(jax.lax.div(q, jnp.int32(4)), 1), :]
    off = jax.lax.rem(q, jnp.int32(4))
    sel = jnp.where(off == 0, row16[:, 0:4],
          jnp.where(off == 1, row16[:, 4:8],
          jnp.where(off == 2, row16[:, 8:12], row16[:, 12:16])))
    cx, cy, w, h = sel[:, 0:1], sel[:, 1:2], sel[:, 2:3], sel[:, 3:4]
    out4 = jnp.concatenate(
        [(cx - 0.5 * w) * img_w, (cy - 0.5 * h) * img_h,
         (cx + 0.5 * w) * img_w, (cy + 0.5 * h) * img_h], axis=1)
    b_ref[0, pl.ds(k, 1), :] = out4
    return jnp.where(flatid == p, NEG, cand)

  jax.lax.fori_loop(0, NSEL, p3, cand_ref[:, :])


def kernel(pred_logits, pred_boxes, target_sizes):
  B, N, C = pred_logits.shape
  flat_n = N * C
  rows = math.ceil(flat_n / 128)
  n_chunks = math.ceil(rows / 128)
  rows_p = n_chunks * 128
  pad = rows_p * 128 - flat_n

  flat = pred_logits.reshape(B, flat_n)
  xp = jnp.pad(flat, ((0, 0), (0, pad)), constant_values=-jnp.inf)
  x3 = xp.reshape(B, rows_p, 128)
  box2 = pred_boxes.reshape(B, (N * 4) // 16, 16)
  ts3 = target_sizes.reshape(B, 1, 2)

  body = functools.partial(_body, rows_p=rows_p, n_chunks=n_chunks,
                           c_dim=C)

  s_out, l_out, b_out, oh_out = pl.pallas_call(
      body,
      grid=(B,),
      in_specs=[
          pl.BlockSpec((1, rows_p, 128), lambda b: (b, 0, 0)),
          pl.BlockSpec((1, (N * 4) // 16, 16), lambda b: (b, 0, 0)),
          pl.BlockSpec((1, 1, 2), lambda b: (b, 0, 0)),
      ],
      out_specs=[
          pl.BlockSpec((1, 128, 1), lambda b: (b, 0, 0)),
          pl.BlockSpec((1, 128, 1), lambda b: (b, 0, 0)),
          pl.BlockSpec((1, 128, 4), lambda b: (b, 0, 0)),
          pl.BlockSpec((1, 128, 128), lambda b: (b, 0, 0)),
      ],
      out_shape=[
          jax.ShapeDtypeStruct((B, 128, 1), jnp.float32),
          jax.ShapeDtypeStruct((B, 128, 1), jnp.int32),
          jax.ShapeDtypeStruct((B, 128, 4), jnp.float32),
          jax.ShapeDtypeStruct((B, 128, 128), jnp.float32),
      ],
      scratch_shapes=[
          pltpu.VMEM((n_chunks, 128), jnp.float32),
          pltpu.VMEM((NCAND, 128), jnp.float32),
          pltpu.VMEM((NCAND, 1), jnp.int32),
      ],
      compiler_params=pltpu.CompilerParams(
          dimension_semantics=("arbitrary",)),
  )(x3, box2, ts3)

  scores = s_out[:, :NSEL, 0]
  labels = l_out[:, :NSEL, 0]
  boxes = b_out[:, :NSEL, :]
  logits_out = oh_out[:, :NSEL, :C]
  return scores, labels, boxes, logits_out


# speedup vs baseline: 1.9414x; 1.0963x over previous
import functools
import math

import jax
import jax.numpy as jnp
from jax.experimental import pallas as pl
from jax.experimental.pallas import tpu as pltpu

NSEL = 100
NCAND = 128
NEG = float("-inf")
IBIG = 2**30


def _body(x_ref, box_ref, ts_ref, s_ref, l_ref, b_ref, oh_ref,
          rowmax_ref, cand_ref, rid_ref, mval_ref, pidx_ref,
          *, n_chunks, c_dim):
  def p1(c, _):
    tile = x_ref[0, pl.ds(c * 128, 128), :]
    rowmax_ref[pl.ds(c, 1), :] = jnp.max(tile, axis=1).reshape(1, 128)
    return 0

  jax.lax.fori_loop(0, n_chunks, p1, 0, unroll=4)

  rid2d = (jax.lax.broadcasted_iota(jnp.int32, (n_chunks, 128), 0) * 128
           + jax.lax.broadcasted_iota(jnp.int32, (n_chunks, 128), 1))

  def p2a(k, rm):
    m = jnp.max(rm, axis=(0, 1), keepdims=True)
    r = jnp.min(jnp.where(rm == m, rid2d, IBIG), axis=(0, 1), keepdims=True)
    rid_ref[pl.ds(k, 1), :] = r
    return jnp.where(rid2d == r, NEG, rm)

  jax.lax.fori_loop(0, NCAND, p2a, rowmax_ref[:, :])

  def p2b(k, _):
    r = rid_ref[k, 0]
    cand_ref[pl.ds(k, 1), :] = x_ref[0, pl.ds(r, 1), :]
    return 0

  jax.lax.fori_loop(0, NCAND, p2b, 0, unroll=4)

  flatid = (rid_ref[:, :] * 128
            + jax.lax.broadcasted_iota(jnp.int32, (NCAND, 128), 1))

  def p3a(k, cand):
    m = jnp.max(cand, axis=(0, 1), keepdims=True)
    p = jnp.min(jnp.where(cand == m, flatid, IBIG), axis=(0, 1),
                keepdims=True)
    mval_ref[pl.ds(k, 1), :] = m
    pidx_ref[pl.ds(k, 1), :] = p
    return jnp.where(flatid == p, NEG, cand)

  jax.lax.fori_loop(0, NSEL, p3a, cand_ref[:, :])

  mv = mval_ref[:, :]
  pv = pidx_ref[:, :]
  s_ref[0, :, :] = 1.0 / (1.0 + jnp.exp(-mv))
  labels = jax.lax.rem(pv, jnp.int32(c_dim))
  l_ref[0, :, :] = labels
  lane128 = jax.lax.broadcasted_iota(jnp.int32, (NCAND, 128), 1)
  oh_ref[0, :, :] = jnp.where(lane128 == labels, 1.0, 0.0)

  img_h = ts_ref[0, 0, 0]
  img_w = ts_ref[0, 0, 1]

  def p3c(k, _):
    p = pidx_ref[k, 0]
    q = jax.lax.div(p, jnp.int32(c_dim))
    row16 = box_ref[0, pl.ds(jax.lax.div(q, jnp.int32(4)), 1), :]
    off = jax.lax.rem(q, jnp.int32(4))
    sel = jnp.where(off == 0, row16[:, 0:4],
          jnp.where(off == 1, row16[:, 4:8],
          jnp.where(off == 2, row16[:, 8:12], row16[:, 12:16])))
    cx, cy, w, h = sel[:, 0:1], sel[:, 1:2], sel[:, 2:3], sel[:, 3:4]
    b_ref[0, pl.ds(k, 1), :] = jnp.concatenate(
        [(cx - 0.5 * w) * img_w, (cy - 0.5 * h) * img_h,
         (cx + 0.5 * w) * img_w, (cy + 0.5 * h) * img_h], axis=1)
    return 0

  jax.lax.fori_loop(0, NSEL, p3c, 0, unroll=4)


def kernel(pred_logits, pred_boxes, target_sizes):
  B, N, C = pred_logits.shape
  flat_n = N * C
  rows = math.ceil(flat_n / 128)
  n_chunks = math.ceil(rows / 128)
  rows_p = n_chunks * 128
  pad = rows_p * 128 - flat_n

  flat = pred_logits.reshape(B, flat_n)
  xp = jnp.pad(flat, ((0, 0), (0, pad)), constant_values=-jnp.inf)
  x3 = xp.reshape(B, rows_p, 128)
  box2 = pred_boxes.reshape(B, (N * 4) // 16, 16)
  ts3 = target_sizes.reshape(B, 1, 2)

  body = functools.partial(_body, n_chunks=n_chunks, c_dim=C)

  s_out, l_out, b_out, oh_out = pl.pallas_call(
      body,
      grid=(B,),
      in_specs=[
          pl.BlockSpec((1, rows_p, 128), lambda b: (b, 0, 0)),
          pl.BlockSpec((1, (N * 4) // 16, 16), lambda b: (b, 0, 0)),
          pl.BlockSpec((1, 1, 2), lambda b: (b, 0, 0)),
      ],
      out_specs=[
          pl.BlockSpec((1, NCAND, 1), lambda b: (b, 0, 0)),
          pl.BlockSpec((1, NCAND, 1), lambda b: (b, 0, 0)),
          pl.BlockSpec((1, NCAND, 4), lambda b: (b, 0, 0)),
          pl.BlockSpec((1, NCAND, 128), lambda b: (b, 0, 0)),
      ],
      out_shape=[
          jax.ShapeDtypeStruct((B, NCAND, 1), jnp.float32),
          jax.ShapeDtypeStruct((B, NCAND, 1), jnp.int32),
          jax.ShapeDtypeStruct((B, NCAND, 4), jnp.float32),
          jax.ShapeDtypeStruct((B, NCAND, 128), jnp.float32),
      ],
      scratch_shapes=[
          pltpu.VMEM((n_chunks, 128), jnp.float32),
          pltpu.VMEM((NCAND, 128), jnp.float32),
          pltpu.VMEM((NCAND, 1), jnp.int32),
          pltpu.VMEM((NCAND, 1), jnp.float32),
          pltpu.VMEM((NCAND, 1), jnp.int32),
      ],
      compiler_params=pltpu.CompilerParams(
          dimension_semantics=("parallel",)),
  )(x3, box2, ts3)

  scores = s_out[:, :NSEL, 0]
  labels = l_out[:, :NSEL, 0]
  boxes = b_out[:, :NSEL, :]
  logits_out = oh_out[:, :NSEL, :C]
  return scores, labels, boxes, logits_out


# speedup vs baseline: 5.4653x; 2.8152x over previous
import functools
import math

import jax
import jax.numpy as jnp
from jax.experimental import pallas as pl
from jax.experimental.pallas import tpu as pltpu

NSEL = 100
NCAND = 128
NEG = float("-inf")
IBIG = 2**30


def _body(x_ref, box_ref, ts_ref, s_ref, l_ref, b_ref, oh_ref,
          rowmax_ref, cand_ref, rid_ref, mval_ref, pidx_ref,
          *, n_full, tail, n_scan, c_dim):
  rowmax_ref[:, :] = jnp.full((n_scan, 128), NEG, jnp.float32)

  def p1(c, _):
    tile = x_ref[0, pl.ds(c * 128, 128), :]
    rowmax_ref[pl.ds(c, 1), :] = jnp.max(tile, axis=1).reshape(1, 128)
    return 0

  jax.lax.fori_loop(0, n_full, p1, 0, unroll=4)
  if tail:
    ttile = x_ref[0, pl.ds(n_full * 128, tail), :]
    rowmax_ref[pl.ds(n_full, 1), 0:tail] = (
        jnp.max(ttile, axis=1).reshape(1, tail))

  rid2d = (jax.lax.broadcasted_iota(jnp.int32, (n_scan, 128), 0) * 128
           + jax.lax.broadcasted_iota(jnp.int32, (n_scan, 128), 1))

  def p2a(k, rm):
    m = jnp.max(rm, axis=(0, 1), keepdims=True)
    r = jnp.min(jnp.where(rm == m, rid2d, IBIG), axis=(0, 1), keepdims=True)
    rid_ref[pl.ds(k, 1), :] = r
    return jnp.where(rid2d == r, NEG, rm)

  jax.lax.fori_loop(0, NCAND, p2a, rowmax_ref[:, :])

  cand_ref[:, :] = jnp.full((NCAND, 128), NEG, jnp.float32)

  def p2b(k, _):
    r = rid_ref[k, 0]
    cand_ref[pl.ds(k, 1), 0:c_dim] = x_ref[0, pl.ds(r, 1), :]
    return 0

  jax.lax.fori_loop(0, NCAND, p2b, 0, unroll=4)

  flatid = (rid_ref[:, :] * c_dim
            + jax.lax.broadcasted_iota(jnp.int32, (NCAND, 128), 1))

  def p3a(k, cand):
    m = jnp.max(cand, axis=(0, 1), keepdims=True)
    p = jnp.min(jnp.where(cand == m, flatid, IBIG), axis=(0, 1),
                keepdims=True)
    mval_ref[pl.ds(k, 1), :] = m
    pidx_ref[pl.ds(k, 1), :] = p
    return jnp.where(flatid == p, NEG, cand)

  jax.lax.fori_loop(0, NSEL, p3a, cand_ref[:, :])

  mv = mval_ref[:, :]
  pv = pidx_ref[:, :]
  s_ref[0, :, :] = 1.0 / (1.0 + jnp.exp(-mv))
  labels = jax.lax.rem(pv, jnp.int32(c_dim))
  l_ref[0, :, :] = labels
  lane128 = jax.lax.broadcasted_iota(jnp.int32, (NCAND, 128), 1)
  oh_ref[0, :, :] = jnp.where(lane128 == labels, 1.0, 0.0)

  img_h = ts_ref[0, 0, 0]
  img_w = ts_ref[0, 0, 1]

  def p3c(k, _):
    p = pidx_ref[k, 0]
    q = jax.lax.div(p, jnp.int32(c_dim))
    row16 = box_ref[0, pl.ds(jax.lax.div(q, jnp.int32(4)), 1), :]
    off = jax.lax.rem(q, jnp.int32(4))
    sel = jnp.where(off == 0, row16[:, 0:4],
          jnp.where(off == 1, row16[:, 4:8],
          jnp.where(off == 2, row16[:, 8:12], row16[:, 12:16])))
    cx, cy, w, h = sel[:, 0:1], sel[:, 1:2], sel[:, 2:3], sel[:, 3:4]
    b_ref[0, pl.ds(k, 1), :] = jnp.concatenate(
        [(cx - 0.5 * w) * img_w, (cy - 0.5 * h) * img_h,
         (cx + 0.5 * w) * img_w, (cy + 0.5 * h) * img_h], axis=1)
    return 0

  jax.lax.fori_loop(0, NSEL, p3c, 0, unroll=4)


def kernel(pred_logits, pred_boxes, target_sizes):
  B, N, C = pred_logits.shape
  assert N >= NCAND and N % 8 == 0 and (N * 4) % 16 == 0
  n_full = N // 128
  tail = N % 128
  n_scan = math.ceil(math.ceil(N / 128) / 8) * 8

  box2 = pred_boxes.reshape(B, (N * 4) // 16, 16)
  ts3 = target_sizes.reshape(B, 1, 2)

  body = functools.partial(_body, n_full=n_full, tail=tail,
                           n_scan=n_scan, c_dim=C)

  s_out, l_out, b_out, oh_out = pl.pallas_call(
      body,
      grid=(B,),
      in_specs=[
          pl.BlockSpec((1, N, C), lambda b: (b, 0, 0)),
          pl.BlockSpec((1, (N * 4) // 16, 16), lambda b: (b, 0, 0)),
          pl.BlockSpec((1, 1, 2), lambda b: (b, 0, 0)),
      ],
      out_specs=[
          pl.BlockSpec((1, NCAND, 1), lambda b: (b, 0, 0)),
          pl.BlockSpec((1, NCAND, 1), lambda b: (b, 0, 0)),
          pl.BlockSpec((1, NCAND, 4), lambda b: (b, 0, 0)),
          pl.BlockSpec((1, NCAND, 128), lambda b: (b, 0, 0)),
      ],
      out_shape=[
          jax.ShapeDtypeStruct((B, NCAND, 1), jnp.float32),
          jax.ShapeDtypeStruct((B, NCAND, 1), jnp.int32),
          jax.ShapeDtypeStruct((B, NCAND, 4), jnp.float32),
          jax.ShapeDtypeStruct((B, NCAND, 128), jnp.float32),
      ],
      scratch_shapes=[
          pltpu.VMEM((n_scan, 128), jnp.float32),
          pltpu.VMEM((NCAND, 128), jnp.float32),
          pltpu.VMEM((NCAND, 1), jnp.int32),
          pltpu.VMEM((NCAND, 1), jnp.float32),
          pltpu.VMEM((NCAND, 1), jnp.int32),
      ],
      compiler_params=pltpu.CompilerParams(
          dimension_semantics=("parallel",)),
  )(pred_logits, box2, ts3)

  scores = s_out[:, :NSEL, 0]
  labels = l_out[:, :NSEL, 0]
  boxes = b_out[:, :NSEL, :]
  logits_out = oh_out[:, :NSEL, :C]
  return scores, labels, boxes, logits_out


# speedup vs baseline: 6.3624x; 1.1641x over previous
import functools
import math

import jax
import jax.numpy as jnp
from jax.experimental import pallas as pl
from jax.experimental.pallas import tpu as pltpu

NSEL = 100
NCAND = 128
NEG = float("-inf")
IBIG = 2**30


def _body(x_ref, box_ref, ts_ref, s_ref, l_ref, b_ref, oh_ref,
          rowmax_ref, cand_ref, rid_ref, mval_ref, pidx_ref,
          *, n_full, tail, n_scan, c_dim):
  rowmax_ref[:, :] = jnp.full((n_scan, 128), NEG, jnp.float32)

  def p1(c, _):
    tile = x_ref[0, pl.ds(c * 128, 128), :]
    rowmax_ref[pl.ds(c, 1), :] = jnp.max(tile, axis=1).reshape(1, 128)
    return 0

  jax.lax.fori_loop(0, n_full, p1, 0, unroll=4)
  if tail:
    ttile = x_ref[0, pl.ds(n_full * 128, tail), :]
    rowmax_ref[pl.ds(n_full, 1), 0:tail] = (
        jnp.max(ttile, axis=1).reshape(1, tail))

  rid2d = (jax.lax.broadcasted_iota(jnp.int32, (n_scan, 128), 0) * 128
           + jax.lax.broadcasted_iota(jnp.int32, (n_scan, 128), 1))

  def p2a(k, rm):
    cm = jnp.max(rm, axis=0, keepdims=True)
    m = jnp.max(cm, axis=1, keepdims=True)
    rl = jnp.min(jnp.where(rm == m, rid2d, IBIG), axis=0, keepdims=True)
    r = jnp.min(rl, axis=1, keepdims=True)
    rid_ref[pl.ds(k, 1), :] = r
    return jnp.where(rid2d == r, NEG, rm)

  jax.lax.fori_loop(0, NCAND, p2a, rowmax_ref[:, :], unroll=2)

  cand_ref[:, :] = jnp.full((NCAND, 128), NEG, jnp.float32)

  def p2b(k, _):
    r = rid_ref[k, 0]
    cand_ref[pl.ds(k, 1), 0:c_dim] = x_ref[0, pl.ds(r, 1), :]
    return 0

  jax.lax.fori_loop(0, NCAND, p2b, 0, unroll=4)

  flatid = (rid_ref[:, :] * c_dim
            + jax.lax.broadcasted_iota(jnp.int32, (NCAND, 128), 1))

  def p3a(k, cand):
    cm = jnp.max(cand, axis=0, keepdims=True)
    m = jnp.max(cm, axis=1, keepdims=True)
    pl_ = jnp.min(jnp.where(cand == m, flatid, IBIG), axis=0, keepdims=True)
    p = jnp.min(pl_, axis=1, keepdims=True)
    mval_ref[pl.ds(k, 1), :] = m
    pidx_ref[pl.ds(k, 1), :] = p
    return jnp.where(flatid == p, NEG, cand)

  jax.lax.fori_loop(0, NSEL, p3a, cand_ref[:, :], unroll=2)

  mv = mval_ref[:, :]
  pv = pidx_ref[:, :]
  s_ref[0, :, :] = 1.0 / (1.0 + jnp.exp(-mv))
  labels = jax.lax.rem(pv, jnp.int32(c_dim))
  l_ref[0, :, :] = labels
  lane128 = jax.lax.broadcasted_iota(jnp.int32, (NCAND, 128), 1)
  oh_ref[0, :, :] = jnp.where(lane128 == labels, 1.0, 0.0)

  img_h = ts_ref[0, 0, 0]
  img_w = ts_ref[0, 0, 1]

  def p3c(k, _):
    p = pidx_ref[k, 0]
    q = jax.lax.div(p, jnp.int32(c_dim))
    row16 = box_ref[0, pl.ds(jax.lax.div(q, jnp.int32(4)), 1), :]
    off = jax.lax.rem(q, jnp.int32(4))
    sel = jnp.where(off == 0, row16[:, 0:4],
          jnp.where(off == 1, row16[:, 4:8],
          jnp.where(off == 2, row16[:, 8:12], row16[:, 12:16])))
    cx, cy, w, h = sel[:, 0:1], sel[:, 1:2], sel[:, 2:3], sel[:, 3:4]
    b_ref[0, pl.ds(k, 1), :] = jnp.concatenate(
        [(cx - 0.5 * w) * img_w, (cy - 0.5 * h) * img_h,
         (cx + 0.5 * w) * img_w, (cy + 0.5 * h) * img_h], axis=1)
    return 0

  jax.lax.fori_loop(0, NSEL, p3c, 0, unroll=4)


def kernel(pred_logits, pred_boxes, target_sizes):
  B, N, C = pred_logits.shape
  assert N >= NCAND and N % 8 == 0 and (N * 4) % 16 == 0
  n_full = N // 128
  tail = N % 128
  n_scan = math.ceil(math.ceil(N / 128) / 8) * 8

  box2 = pred_boxes.reshape(B, (N * 4) // 16, 16)
  ts3 = target_sizes.reshape(B, 1, 2)

  body = functools.partial(_body, n_full=n_full, tail=tail,
                           n_scan=n_scan, c_dim=C)

  s_out, l_out, b_out, oh_out = pl.pallas_call(
      body,
      grid=(B,),
      in_specs=[
          pl.BlockSpec((1, N, C), lambda b: (b, 0, 0)),
          pl.BlockSpec((1, (N * 4) // 16, 16), lambda b: (b, 0, 0)),
          pl.BlockSpec((1, 1, 2), lambda b: (b, 0, 0)),
      ],
      out_specs=[
          pl.BlockSpec((1, NCAND, 1), lambda b: (b, 0, 0)),
          pl.BlockSpec((1, NCAND, 1), lambda b: (b, 0, 0)),
          pl.BlockSpec((1, NCAND, 4), lambda b: (b, 0, 0)),
          pl.BlockSpec((1, NCAND, 128), lambda b: (b, 0, 0)),
      ],
      out_shape=[
          jax.ShapeDtypeStruct((B, NCAND, 1), jnp.float32),
          jax.ShapeDtypeStruct((B, NCAND, 1), jnp.int32),
          jax.ShapeDtypeStruct((B, NCAND, 4), jnp.float32),
          jax.ShapeDtypeStruct((B, NCAND, 128), jnp.float32),
      ],
      scratch_shapes=[
          pltpu.VMEM((n_scan, 128), jnp.float32),
          pltpu.VMEM((NCAND, 128), jnp.float32),
          pltpu.VMEM((NCAND, 1), jnp.int32),
          pltpu.VMEM((NCAND, 1), jnp.float32),
          pltpu.VMEM((NCAND, 1), jnp.int32),
      ],
      compiler_params=pltpu.CompilerParams(
          dimension_semantics=("parallel",)),
  )(pred_logits, box2, ts3)

  scores = s_out[:, :NSEL, 0]
  labels = l_out[:, :NSEL, 0]
  boxes = b_out[:, :NSEL, :]
  logits_out = oh_out[:, :NSEL, :C]
  return scores, labels, boxes, logits_out


# speedup vs baseline: 6.6760x; 1.0493x over previous
import functools
import math

import jax
import jax.numpy as jnp
from jax.experimental import pallas as pl
from jax.experimental.pallas import tpu as pltpu

NSEL = 100
NCAND = 112
NEG = float("-inf")
IBIG = 2**30


def _body(x_ref, box_ref, ts_ref, s_ref, l_ref, b_ref, oh_ref,
          rowmax_ref, cand_ref, rid_ref, mval_ref, pidx_ref,
          *, n_full, tail, n_scan, c_dim):
  rowmax_ref[:, :] = jnp.full((n_scan, 128), NEG, jnp.float32)

  def p1(c, _):
    tile = x_ref[0, pl.ds(c * 128, 128), :]
    rowmax_ref[pl.ds(c, 1), :] = jnp.max(tile, axis=1).reshape(1, 128)
    return 0

  jax.lax.fori_loop(0, n_full, p1, 0, unroll=4)
  if tail:
    ttile = x_ref[0, pl.ds(n_full * 128, tail), :]
    rowmax_ref[pl.ds(n_full, 1), 0:tail] = (
        jnp.max(ttile, axis=1).reshape(1, tail))

  rid2d = (jax.lax.broadcasted_iota(jnp.int32, (n_scan, 128), 0) * 128
           + jax.lax.broadcasted_iota(jnp.int32, (n_scan, 128), 1))

  def p2a(k, rm):
    cm = jnp.max(rm, axis=0, keepdims=True)
    m = jnp.max(cm, axis=1, keepdims=True)
    rl = jnp.min(jnp.where(rm == m, rid2d, IBIG), axis=0, keepdims=True)
    r = jnp.min(rl, axis=1, keepdims=True)
    rid_ref[pl.ds(k, 1), :] = r
    return jnp.where(rid2d == r, NEG, rm)

  jax.lax.fori_loop(0, NCAND, p2a, rowmax_ref[:, :], unroll=2)

  cand_ref[:, :] = jnp.full((NCAND, 128), NEG, jnp.float32)

  def p2b(k, _):
    r = rid_ref[k, 0]
    cand_ref[pl.ds(k, 1), 0:c_dim] = x_ref[0, pl.ds(r, 1), :]
    return 0

  jax.lax.fori_loop(0, NCAND, p2b, 0, unroll=4)

  flatid = (rid_ref[:, :] * c_dim
            + jax.lax.broadcasted_iota(jnp.int32, (NCAND, 128), 1))

  def p3a(k, cand):
    cm = jnp.max(cand, axis=0, keepdims=True)
    m = jnp.max(cm, axis=1, keepdims=True)
    pl_ = jnp.min(jnp.where(cand == m, flatid, IBIG), axis=0, keepdims=True)
    p = jnp.min(pl_, axis=1, keepdims=True)
    mval_ref[pl.ds(k, 1), :] = m
    pidx_ref[pl.ds(k, 1), :] = p
    return jnp.where(flatid == p, NEG, cand)

  jax.lax.fori_loop(0, NSEL, p3a, cand_ref[:, :], unroll=2)

  mv = mval_ref[:, :]
  pv = pidx_ref[:, :]
  s_ref[0, :, :] = 1.0 / (1.0 + jnp.exp(-mv))
  labels = jax.lax.rem(pv, jnp.int32(c_dim))
  l_ref[0, :, :] = labels
  lane128 = jax.lax.broadcasted_iota(jnp.int32, (NCAND, 128), 1)
  oh_ref[0, :, :] = jnp.where(lane128 == labels, 1.0, 0.0)

  img_h = ts_ref[0, 0, 0]
  img_w = ts_ref[0, 0, 1]

  def p3c(k, _):
    p = pidx_ref[k, 0]
    q = jax.lax.div(p, jnp.int32(c_dim))
    row16 = box_ref[0, pl.ds(jax.lax.div(q, jnp.int32(4)), 1), :]
    off = jax.lax.rem(q, jnp.int32(4))
    sel = jnp.where(off == 0, row16[:, 0:4],
          jnp.where(off == 1, row16[:, 4:8],
          jnp.where(off == 2, row16[:, 8:12], row16[:, 12:16])))
    cx, cy, w, h = sel[:, 0:1], sel[:, 1:2], sel[:, 2:3], sel[:, 3:4]
    b_ref[0, pl.ds(k, 1), :] = jnp.concatenate(
        [(cx - 0.5 * w) * img_w, (cy - 0.5 * h) * img_h,
         (cx + 0.5 * w) * img_w, (cy + 0.5 * h) * img_h], axis=1)
    return 0

  jax.lax.fori_loop(0, NSEL, p3c, 0, unroll=4)


def kernel(pred_logits, pred_boxes, target_sizes):
  B, N, C = pred_logits.shape
  assert N >= NCAND and N % 8 == 0 and (N * 4) % 16 == 0
  n_full = N // 128
  tail = N % 128
  n_scan = math.ceil(math.ceil(N / 128) / 8) * 8

  box2 = pred_boxes.reshape(B, (N * 4) // 16, 16)
  ts3 = target_sizes.reshape(B, 1, 2)

  body = functools.partial(_body, n_full=n_full, tail=tail,
                           n_scan=n_scan, c_dim=C)

  s_out, l_out, b_out, oh_out = pl.pallas_call(
      body,
      grid=(B,),
      in_specs=[
          pl.BlockSpec((1, N, C), lambda b: (b, 0, 0)),
          pl.BlockSpec((1, (N * 4) // 16, 16), lambda b: (b, 0, 0)),
          pl.BlockSpec((1, 1, 2), lambda b: (b, 0, 0)),
      ],
      out_specs=[
          pl.BlockSpec((1, NCAND, 1), lambda b: (b, 0, 0)),
          pl.BlockSpec((1, NCAND, 1), lambda b: (b, 0, 0)),
          pl.BlockSpec((1, NCAND, 4), lambda b: (b, 0, 0)),
          pl.BlockSpec((1, NCAND, 128), lambda b: (b, 0, 0)),
      ],
      out_shape=[
          jax.ShapeDtypeStruct((B, NCAND, 1), jnp.float32),
          jax.ShapeDtypeStruct((B, NCAND, 1), jnp.int32),
          jax.ShapeDtypeStruct((B, NCAND, 4), jnp.float32),
          jax.ShapeDtypeStruct((B, NCAND, 128), jnp.float32),
      ],
      scratch_shapes=[
          pltpu.VMEM((n_scan, 128), jnp.float32),
          pltpu.VMEM((NCAND, 128), jnp.float32),
          pltpu.VMEM((NCAND, 1), jnp.int32),
          pltpu.VMEM((NCAND, 1), jnp.float32),
          pltpu.VMEM((NCAND, 1), jnp.int32),
      ],
      compiler_params=pltpu.CompilerParams(
          dimension_semantics=("parallel",)),
  )(pred_logits, box2, ts3)

  scores = s_out[:, :NSEL, 0]
  labels = l_out[:, :NSEL, 0]
  boxes = b_out[:, :NSEL, :]
  logits_out = oh_out[:, :NSEL, :C]
  return scores, labels, boxes, logits_out
